# baseline (speedup 1.0000x reference)
import functools

import jax
import jax.numpy as jnp
from jax.experimental import pallas as pl


B, D, H, R = 16384, 2048, 128, 16
TB = 512


def _router_kernel(x_ref, w1_ref, b1_ref, w2_ref, b2_ref, rb_ref,
                   sel_ref, prob_ref):
    x = x_ref[...]
    h = jnp.maximum(
        jnp.dot(x, w1_ref[...], preferred_element_type=jnp.float32)
        + b1_ref[...], 0.0)
    logits = (jnp.dot(h, w2_ref[...], preferred_element_type=jnp.float32)
              + b2_ref[...] + rb_ref[...])
    m = jnp.max(logits, axis=-1, keepdims=True)
    e = jnp.exp(logits - m)
    probs = e / jnp.sum(e, axis=-1, keepdims=True)
    prob_ref[...] = probs
    sel_ref[...] = jnp.argmax(probs, axis=-1).astype(jnp.int32)


@functools.partial(jax.jit, static_argnames=())
def kernel(x, W1, b1, W2, b2, route_bias):
    grid = (B // TB,)
    sel, probs = pl.pallas_call(
        _router_kernel,
        grid=grid,
        in_specs=[
            pl.BlockSpec((TB, D), lambda i: (i, 0)),
            pl.BlockSpec((D, H), lambda i: (0, 0)),
            pl.BlockSpec((1, H), lambda i: (0, 0)),
            pl.BlockSpec((H, R), lambda i: (0, 0)),
            pl.BlockSpec((1, R), lambda i: (0, 0)),
            pl.BlockSpec((1, R), lambda i: (0, 0)),
        ],
        out_specs=[
            pl.BlockSpec((TB,), lambda i: (i,)),
            pl.BlockSpec((TB, R), lambda i: (i, 0)),
        ],
        out_shape=[
            jax.ShapeDtypeStruct((B,), jnp.int32),
            jax.ShapeDtypeStruct((B, R), jnp.float32),
        ],
    )(x, W1, b1.reshape(1, H), W2, b2.reshape(1, R),
      route_bias.reshape(1, R))
    return (sel, probs)
